# Initial kernel scaffold; baseline (speedup 1.0000x reference)
import jax
import jax.numpy as jnp
from jax.experimental import pallas as pl


def kernel(x, edge_index, W1a, b1a, W1b, b1b, g1, be1, W2a, b2a, W2b, b2b, g2, be2):
    raise NotImplementedError("write your pallas kernel here")


# speedup vs baseline: 2.8429x; 2.8429x over previous
import functools

import jax
import jax.numpy as jnp
from jax import lax
from jax.experimental import pallas as pl
from jax.experimental.pallas import tpu as pltpu
from jax.experimental.pallas import tpu_sc as plsc

N = 10000
D = 128
E = 320000
EPS_BN = 1e-5

NC = 2
NS = 16
NW = NC * NS
CHUNK = 128
CPW = 80
EP = NW * CHUNK * CPW
RPT = 624
TAIL = N - NS * RPT
ACC_ROWS = 10016

_sc_mesh = plsc.VectorSubcoreMesh(core_axis_name="c", subcore_axis_name="s")


@functools.partial(
    pl.kernel,
    mesh=_sc_mesh,
    out_type=jax.ShapeDtypeStruct((2 * N, D), jnp.float32),
    scratch_types=[
        pltpu.VMEM_SHARED((ACC_ROWS, D), jnp.float32),
        pltpu.VMEM((CPW, CHUNK), jnp.int32),
        pltpu.VMEM((CPW, CHUNK), jnp.int32),
        pltpu.VMEM((CHUNK, D), jnp.float32),
        pltpu.SemaphoreType.DMA,
    ],
)
def _sc_agg(x_hbm, src_hbm, dst_hbm, out_hbm, acc, src_idx, dst_idx, rb, sem):
    c = lax.axis_index("c")
    s = lax.axis_index("s")
    wid = c * NS + s
    pltpu.sync_copy(x_hbm.at[pl.ds(s * RPT, RPT)], acc.at[pl.ds(s * RPT, RPT)])

    @pl.when(s == NS - 1)
    def _():
        pltpu.sync_copy(x_hbm.at[pl.ds(NS * RPT, TAIL)],
                        acc.at[pl.ds(NS * RPT, TAIL)])
    pltpu.sync_copy(src_hbm.at[pl.ds(wid * CPW, CPW)], src_idx)
    pltpu.sync_copy(dst_hbm.at[pl.ds(wid * CPW, CPW)], dst_idx)
    plsc.subcore_barrier()

    def body(i, carry):
        pltpu.async_copy(x_hbm.at[src_idx.at[i]], rb, sem).wait()
        pltpu.sync_copy(rb, acc.at[dst_idx.at[i]], add=True)
        return carry

    lax.fori_loop(0, CPW, body, 0)
    plsc.subcore_barrier()
    pltpu.sync_copy(acc.at[pl.ds(s * RPT, RPT)],
                    out_hbm.at[pl.ds(c * N + s * RPT, RPT)])

    @pl.when(s == NS - 1)
    def _():
        pltpu.sync_copy(acc.at[pl.ds(NS * RPT, TAIL)],
                        out_hbm.at[pl.ds(c * N + NS * RPT, TAIL)])


def _tc_mlp_body(x_ref, p_ref, wa_ref, ba_ref, wb_ref, bb_ref, g_ref, be_ref, o_ref):
    h = p_ref[0:N, :] + p_ref[N:2 * N, :] - x_ref[...]
    h = jnp.maximum(jnp.dot(h, wa_ref[...], preferred_element_type=jnp.float32)
                    + ba_ref[...], 0.0)
    h = jnp.maximum(jnp.dot(h, wb_ref[...], preferred_element_type=jnp.float32)
                    + bb_ref[...], 0.0)
    mean = jnp.mean(h, axis=0, keepdims=True)
    zc = h - mean
    var = jnp.mean(zc * zc, axis=0, keepdims=True)
    o_ref[...] = zc * lax.rsqrt(var + EPS_BN) * g_ref[...] + be_ref[...]


_tc_mlp = pl.pallas_call(
    _tc_mlp_body,
    out_shape=jax.ShapeDtypeStruct((N, D), jnp.float32),
)


def kernel(x, edge_index, W1a, b1a, W1b, b1b, g1, be1, W2a, b2a, W2b, b2b, g2, be2):
    pad = EP - E
    src = jnp.concatenate([edge_index[0], jnp.zeros((pad,), jnp.int32)])
    dst = jnp.concatenate([edge_index[1], jnp.full((pad,), N, jnp.int32)])
    src2 = src.reshape(NW * CPW, CHUNK)
    dst2 = dst.reshape(NW * CPW, CHUNK)

    def layer(h, wa, ba, wb, bb, g, be):
        parts = _sc_agg(h, src2, dst2)
        return _tc_mlp(h, parts, wa, ba.reshape(1, D), wb, bb.reshape(1, D),
                       g.reshape(1, D), be.reshape(1, D))

    h1 = layer(x, W1a, b1a, W1b, b1b, g1, be1)
    return layer(h1, W2a, b2a, W2b, b2b, g2, be2)


# speedup vs baseline: 9.8692x; 3.4716x over previous
import functools

import jax
import jax.numpy as jnp
from jax import lax
from jax.experimental import pallas as pl
from jax.experimental.pallas import tpu as pltpu
from jax.experimental.pallas import tpu_sc as plsc

N = 10000
D = 128
E = 320000
EPS_BN = 1e-5

NC = 2
NS = 16
NW = NC * NS
CHUNK = 128
CPW = 80
EP = NW * CHUNK * CPW
NREAL = E // CHUNK
RPT = 624
TAIL = N - NS * RPT
ACC_ROWS = 10016

_sc_mesh = plsc.VectorSubcoreMesh(core_axis_name="c", subcore_axis_name="s")


@functools.partial(
    pl.kernel,
    mesh=_sc_mesh,
    out_type=jax.ShapeDtypeStruct((2 * N, D), jnp.float32),
    scratch_types=[
        pltpu.VMEM_SHARED((ACC_ROWS, D), jnp.float32),
        pltpu.VMEM((CPW // 2, CHUNK), jnp.int32),
        pltpu.VMEM((CPW // 2, CHUNK), jnp.int32),
        pltpu.VMEM((CHUNK, D), jnp.float32),
        pltpu.VMEM((CHUNK, D), jnp.float32),
        pltpu.SemaphoreType.DMA,
        pltpu.SemaphoreType.DMA,
        pltpu.SemaphoreType.DMA,
        pltpu.SemaphoreType.DMA,
    ],
)
def _sc_agg(x_hbm, src_hbm, dst_hbm, out_hbm, acc, src_idx, dst_idx,
            rb0, rb1, gs0, gs1, ss0, ss1):
    c = lax.axis_index("c")
    s = lax.axis_index("s")
    wid = c * NS + s
    pltpu.sync_copy(x_hbm.at[pl.ds(s * RPT, RPT)], acc.at[pl.ds(s * RPT, RPT)])

    @pl.when(s == NS - 1)
    def _():
        pltpu.sync_copy(x_hbm.at[pl.ds(NS * RPT, TAIL)],
                        acc.at[pl.ds(NS * RPT, TAIL)])
    plsc.subcore_barrier()

    HALF = CPW // 2
    for h in range(2):
        hbase = wid * CPW + h * HALF
        pltpu.sync_copy(src_hbm.at[pl.ds(hbase, HALF)], src_idx)
        pltpu.sync_copy(dst_hbm.at[pl.ds(hbase, HALF)], dst_idx)

        def step(j, i, rb, gsem, ssem):
            real = hbase + i < NREAL

            @pl.when(jnp.logical_and(j > 0, real))
            def _():
                pltpu.make_async_copy(rb, acc.at[dst_idx.at[i]], ssem).wait()

            @pl.when(real)
            def _():
                pltpu.async_copy(x_hbm.at[src_idx.at[i]], rb, gsem)

        def fire(i, rb, gsem, ssem):
            real = hbase + i < NREAL

            @pl.when(real)
            def _():
                pltpu.make_async_copy(x_hbm.at[src_idx.at[i]], rb, gsem).wait()
                pltpu.async_copy(rb, acc.at[dst_idx.at[i]], ssem, add=True)

        def body(j, carry):
            i0 = 2 * j
            i1 = i0 + 1
            step(j, i0, rb0, gs0, ss0)
            step(j, i1, rb1, gs1, ss1)
            fire(i0, rb0, gs0, ss0)
            fire(i1, rb1, gs1, ss1)
            return carry

        lax.fori_loop(0, HALF // 2, body, 0)
        @pl.when(hbase < NREAL)
        def _():
            pltpu.make_async_copy(rb0, acc.at[dst_idx.at[0]], ss0).wait()

        @pl.when(hbase + 1 < NREAL)
        def _():
            pltpu.make_async_copy(rb1, acc.at[dst_idx.at[1]], ss1).wait()
    plsc.subcore_barrier()
    pltpu.sync_copy(acc.at[pl.ds(s * RPT, RPT)],
                    out_hbm.at[pl.ds(c * N + s * RPT, RPT)])

    @pl.when(s == NS - 1)
    def _():
        pltpu.sync_copy(acc.at[pl.ds(NS * RPT, TAIL)],
                        out_hbm.at[pl.ds(c * N + NS * RPT, TAIL)])


def _tc_mlp_body(x_ref, p_ref, wa_ref, ba_ref, wb_ref, bb_ref, g_ref, be_ref, o_ref):
    h = p_ref[0:N, :] + p_ref[N:2 * N, :] - x_ref[...]
    h = jnp.maximum(jnp.dot(h, wa_ref[...], preferred_element_type=jnp.float32)
                    + ba_ref[...], 0.0)
    h = jnp.maximum(jnp.dot(h, wb_ref[...], preferred_element_type=jnp.float32)
                    + bb_ref[...], 0.0)
    mean = jnp.mean(h, axis=0, keepdims=True)
    zc = h - mean
    var = jnp.mean(zc * zc, axis=0, keepdims=True)
    o_ref[...] = zc * lax.rsqrt(var + EPS_BN) * g_ref[...] + be_ref[...]


_tc_mlp = pl.pallas_call(
    _tc_mlp_body,
    out_shape=jax.ShapeDtypeStruct((N, D), jnp.float32),
)


def kernel(x, edge_index, W1a, b1a, W1b, b1b, g1, be1, W2a, b2a, W2b, b2b, g2, be2):
    pad = EP - E
    src = jnp.concatenate([edge_index[0], jnp.zeros((pad,), jnp.int32)])
    dst = jnp.concatenate([edge_index[1], jnp.full((pad,), N, jnp.int32)])
    src2 = src.reshape(NW * CPW, CHUNK)
    dst2 = dst.reshape(NW * CPW, CHUNK)

    def layer(h, wa, ba, wb, bb, g, be):
        parts = _sc_agg(h, src2, dst2)
        return _tc_mlp(h, parts, wa, ba.reshape(1, D), wb, bb.reshape(1, D),
                       g.reshape(1, D), be.reshape(1, D))

    h1 = layer(x, W1a, b1a, W1b, b1b, g1, be1)
    return layer(h1, W2a, b2a, W2b, b2b, g2, be2)


# speedup vs baseline: 10.1081x; 1.0242x over previous
import functools

import jax
import jax.numpy as jnp
from jax import lax
from jax.experimental import pallas as pl
from jax.experimental.pallas import tpu as pltpu
from jax.experimental.pallas import tpu_sc as plsc

N = 10000
D = 128
E = 320000
EPS_BN = 1e-5

NC = 2
NS = 16
NW = NC * NS
CHUNK = 128
CPW = 80
EP = NW * CHUNK * CPW
NREAL = E // CHUNK
S = 16
NST = CPW // S
RPS = S // 2
RPT = 624
TAIL = N - NS * RPT
XSPLIT = 4992

_sc_mesh = plsc.VectorSubcoreMesh(core_axis_name="c", subcore_axis_name="s")


@functools.partial(
    pl.kernel,
    mesh=_sc_mesh,
    out_type=jax.ShapeDtypeStruct((2 * N, D), jnp.float32),
    scratch_types=[
        pltpu.VMEM_SHARED((N, D), jnp.float32),
        pltpu.VMEM((S, CHUNK), jnp.int32),
        pltpu.VMEM((S, CHUNK), jnp.int32),
        pltpu.VMEM((S, CHUNK), jnp.int32),
        pltpu.VMEM((S, CHUNK), jnp.int32),
        pltpu.VMEM((CHUNK, D), jnp.float32),
        pltpu.VMEM((CHUNK, D), jnp.float32),
        pltpu.SemaphoreType.DMA,
        pltpu.SemaphoreType.DMA,
        pltpu.SemaphoreType.DMA,
        pltpu.SemaphoreType.DMA,
        pltpu.SemaphoreType.DMA,
        pltpu.SemaphoreType.DMA,
    ],
)
def _sc_agg(x_hbm, src_hbm, dst_hbm, zero_hbm, out_hbm, acc,
            si0, di0, si1, di1, rb0, rb1, gs0, gs1, ss0, ss1, isrc, idst):
    c = lax.axis_index("c")
    s_ax = lax.axis_index("s")
    wid = c * NS + s_ax
    base = wid * CPW
    row0 = s_ax * RPT

    mine = jnp.where(c == 0, row0 < XSPLIT, row0 >= XSPLIT)

    @pl.when(mine)
    def _():
        pltpu.sync_copy(x_hbm.at[pl.ds(row0, RPT)], acc.at[pl.ds(row0, RPT)])

    @pl.when(jnp.logical_not(mine))
    def _():
        pltpu.sync_copy(zero_hbm.at[pl.ds(0, RPT)], acc.at[pl.ds(row0, RPT)])

    @pl.when(s_ax == NS - 1)
    def _():
        tmine = jnp.where(c == 0, NS * RPT < XSPLIT, NS * RPT >= XSPLIT)

        @pl.when(tmine)
        def _():
            pltpu.sync_copy(x_hbm.at[pl.ds(NS * RPT, TAIL)],
                            acc.at[pl.ds(NS * RPT, TAIL)])

        @pl.when(jnp.logical_not(tmine))
        def _():
            pltpu.sync_copy(zero_hbm.at[pl.ds(0, TAIL)],
                            acc.at[pl.ds(NS * RPT, TAIL)])

    plsc.subcore_barrier()

    sbuf = (si0, si1)
    dbuf = (di0, di1)
    pltpu.async_copy(src_hbm.at[pl.ds(base, S)], si0, isrc)
    pltpu.async_copy(dst_hbm.at[pl.ds(base, S)], di0, idst)

    for st in range(NST):
        p = st % 2
        sidx = sbuf[p]
        didx = dbuf[p]
        pltpu.make_async_copy(src_hbm.at[pl.ds(base + st * S, S)], sidx, isrc).wait()
        pltpu.make_async_copy(dst_hbm.at[pl.ds(base + st * S, S)], didx, idst).wait()

        def body(j, carry, st=st, sidx=sidx, didx=didx):
            g0 = base + st * S + 2 * j
            g1 = g0 + 1

            def slot(i_loc, g, rb, gsem, ssem):
                first = (st == 0) & (j == 0) if st == 0 else jnp.bool_(False)

                @pl.when(jnp.logical_and(jnp.logical_not(first), g - 2 < NREAL))
                def _():
                    pltpu.make_async_copy(rb, acc.at[didx.at[i_loc]], ssem).wait()

                @pl.when(g < NREAL)
                def _():
                    pltpu.async_copy(x_hbm.at[sidx.at[i_loc]], rb, gsem)

            def fire(i_loc, g, rb, gsem, ssem):
                @pl.when(g < NREAL)
                def _():
                    pltpu.make_async_copy(x_hbm.at[sidx.at[i_loc]], rb, gsem).wait()
                    pltpu.async_copy(rb, acc.at[didx.at[i_loc]], ssem, add=True)

            slot(2 * j, g0, rb0, gs0, ss0)
            slot(2 * j + 1, g1, rb1, gs1, ss1)
            fire(2 * j, g0, rb0, gs0, ss0)
            fire(2 * j + 1, g1, rb1, gs1, ss1)

            if st < NST - 1:
                @pl.when(j == 1)
                def _():
                    nxt = base + (st + 1) * S
                    pltpu.async_copy(src_hbm.at[pl.ds(nxt, S)], sbuf[1 - p], isrc)
                    pltpu.async_copy(dst_hbm.at[pl.ds(nxt, S)], dbuf[1 - p], idst)

            return carry

        lax.fori_loop(0, RPS, body, 0)

    last0 = base + 2 * (CPW // 2 - 1)

    @pl.when(last0 < NREAL)
    def _():
        pltpu.make_async_copy(rb0, acc.at[dbuf[(NST - 1) % 2].at[0]], ss0).wait()

    @pl.when(last0 + 1 < NREAL)
    def _():
        pltpu.make_async_copy(rb1, acc.at[dbuf[(NST - 1) % 2].at[1]], ss1).wait()

    plsc.subcore_barrier()
    pltpu.sync_copy(acc.at[pl.ds(row0, RPT)],
                    out_hbm.at[pl.ds(c * N + row0, RPT)])

    @pl.when(s_ax == NS - 1)
    def _():
        pltpu.sync_copy(acc.at[pl.ds(NS * RPT, TAIL)],
                        out_hbm.at[pl.ds(c * N + NS * RPT, TAIL)])


def _tc_mlp_body(p_ref, wa_ref, ba_ref, wb_ref, bb_ref, g_ref, be_ref, o_ref):
    h = p_ref[0:N, :] + p_ref[N:2 * N, :]
    h = jnp.maximum(jnp.dot(h, wa_ref[...], preferred_element_type=jnp.float32)
                    + ba_ref[...], 0.0)
    h = jnp.maximum(jnp.dot(h, wb_ref[...], preferred_element_type=jnp.float32)
                    + bb_ref[...], 0.0)
    mean = jnp.mean(h, axis=0, keepdims=True)
    zc = h - mean
    var = jnp.mean(zc * zc, axis=0, keepdims=True)
    o_ref[...] = zc * lax.rsqrt(var + EPS_BN) * g_ref[...] + be_ref[...]


_tc_mlp = pl.pallas_call(
    _tc_mlp_body,
    out_shape=jax.ShapeDtypeStruct((N, D), jnp.float32),
)


def kernel(x, edge_index, W1a, b1a, W1b, b1b, g1, be1, W2a, b2a, W2b, b2b, g2, be2):
    pad = EP - E
    src = jnp.concatenate([edge_index[0], jnp.zeros((pad,), jnp.int32)])
    dst = jnp.concatenate([edge_index[1], jnp.zeros((pad,), jnp.int32)])
    src2 = src.reshape(NW * CPW, CHUNK)
    dst2 = dst.reshape(NW * CPW, CHUNK)
    zeros = jnp.zeros((RPT, D), jnp.float32)

    def layer(h, wa, ba, wb, bb, g, be):
        parts = _sc_agg(h, src2, dst2, zeros)
        return _tc_mlp(parts, wa, ba.reshape(1, D), wb, bb.reshape(1, D),
                       g.reshape(1, D), be.reshape(1, D))

    h1 = layer(x, W1a, b1a, W1b, b1b, g1, be1)
    return layer(h1, W2a, b2a, W2b, b2b, g2, be2)
